# Initial kernel scaffold; baseline (speedup 1.0000x reference)
import jax
import jax.numpy as jnp
from jax.experimental import pallas as pl


def kernel(fmess, bgraph, Wi, bi, Wo, bo, Wf, bf, Wu, bu):
    raise NotImplementedError("write your pallas kernel here")


# speedup vs baseline: 21.7838x; 21.7838x over previous
import functools

import jax
import jax.numpy as jnp
from jax import lax
from jax.experimental import pallas as pl
from jax.experimental.pallas import tpu as pltpu
from jax.experimental.pallas import tpu_sc as plsc

E = 160000
NNEI = 8
IN = 128
HID = 64
F32 = jnp.float32

BE = 2000
NC = 2
NS = 16
NW = NC * NS
PER_W = E // NW
CH = 8
NCHUNK = PER_W // CH
ROWS = CH * NNEI
LANES = 16
NV = HID // LANES
REC = 4 * HID


def _sig(x):
    return 1.0 / (1.0 + jnp.exp(-x))


def _keep_mask(nrows):
    pid = pl.program_id(0)
    rowids = lax.broadcasted_iota(jnp.int32, (nrows, 1), 0) + pid * nrows
    return (rowids != 0).astype(F32)


def _phase_a_body(f_ref, wx_ref, b_ref, wfh_ref, pio_ref, pf_ref, rec_ref):
    pre = jnp.dot(f_ref[...], wx_ref[...], preferred_element_type=F32) + b_ref[...]
    pio_ref[...] = pre[:, :3 * HID]
    pf_ref[...] = pre[:, 3 * HID:]
    i0 = _sig(pre[:, :HID])
    o0 = _sig(pre[:, HID:2 * HID])
    u0 = jnp.tanh(pre[:, 2 * HID:3 * HID])
    keep = _keep_mask(BE)
    c1 = i0 * u0 * keep
    h1 = o0 * jnp.tanh(c1) * keep
    g1 = jnp.dot(h1, wfh_ref[...], preferred_element_type=F32)
    rec_ref[...] = jnp.concatenate([h1, g1, c1, jnp.zeros((BE, HID), F32)], axis=1)


def _phase_c_mid_body(pio_ref, hs_ref, ca_ref, wh_ref, wfh_ref, rec_ref):
    t = pio_ref[...] + jnp.dot(hs_ref[...], wh_ref[...], preferred_element_type=F32)
    i = _sig(t[:, :HID])
    o = _sig(t[:, HID:2 * HID])
    u = jnp.tanh(t[:, 2 * HID:])
    keep = _keep_mask(BE)
    c = (i * u + ca_ref[...]) * keep
    h = o * jnp.tanh(c) * keep
    g = jnp.dot(h, wfh_ref[...], preferred_element_type=F32)
    rec_ref[...] = jnp.concatenate([h, g, c, jnp.zeros((BE, HID), F32)], axis=1)


def _phase_c_final_body(pio_ref, hs_ref, ca_ref, wh_ref, h_ref, c_ref):
    t = pio_ref[...] + jnp.dot(hs_ref[...], wh_ref[...], preferred_element_type=F32)
    i = _sig(t[:, :HID])
    o = _sig(t[:, HID:2 * HID])
    u = jnp.tanh(t[:, 2 * HID:])
    keep = _keep_mask(BE)
    c = (i * u + ca_ref[...]) * keep
    h = o * jnp.tanh(c) * keep
    h_ref[...] = h
    c_ref[...] = c


def _sc_body(rec_hbm, pref_hbm, idx_hbm, hs_hbm, ca_hbm,
             idx_v, rows_v, pref_v, hs_v, ca_v, sem):
    wid = lax.axis_index("s") * NC + lax.axis_index("c")
    base = wid * PER_W

    def chunk_body(ci, carry):
        m0 = base + ci * CH
        pltpu.sync_copy(idx_hbm.at[pl.ds(m0 * NNEI, ROWS)], idx_v)
        pltpu.async_copy(rec_hbm.at[idx_v], rows_v, sem).wait()
        pltpu.sync_copy(pref_hbm.at[pl.ds(m0, CH)], pref_v)

        def msg_body(m, c2):
            r0 = m * NNEI
            for v in range(NV):
                sl = pl.ds(v * LANES, LANES)
                slg = pl.ds(HID + v * LANES, LANES)
                slc = pl.ds(2 * HID + v * LANES, LANES)
                p = pref_v[m, sl]
                hs_acc = rows_v[r0, sl]
                ca_acc = rows_v[r0, slc] / (1.0 + jnp.exp(-(p + rows_v[r0, slg])))
                for k in range(1, NNEI):
                    hs_acc = hs_acc + rows_v[r0 + k, sl]
                    ca_acc = ca_acc + rows_v[r0 + k, slc] / (
                        1.0 + jnp.exp(-(p + rows_v[r0 + k, slg])))
                hs_v[m, sl] = hs_acc
                ca_v[m, sl] = ca_acc
            return c2

        lax.fori_loop(0, CH, msg_body, 0)
        pltpu.sync_copy(hs_v, hs_hbm.at[pl.ds(m0, CH)])
        pltpu.sync_copy(ca_v, ca_hbm.at[pl.ds(m0, CH)])
        return carry

    lax.fori_loop(0, NCHUNK, chunk_body, 0)


@functools.cache
def _sc_gather():
    mesh = plsc.VectorSubcoreMesh(core_axis_name="c", subcore_axis_name="s")
    return pl.kernel(
        _sc_body,
        mesh=mesh,
        out_type=(jax.ShapeDtypeStruct((E, HID), F32),
                  jax.ShapeDtypeStruct((E, HID), F32)),
        scratch_types=[
            pltpu.VMEM((ROWS,), jnp.int32),
            pltpu.VMEM((ROWS, REC), F32),
            pltpu.VMEM((CH, HID), F32),
            pltpu.VMEM((CH, HID), F32),
            pltpu.VMEM((CH, HID), F32),
            pltpu.SemaphoreType.DMA,
        ],
    )


def _rows(w):
    return pl.BlockSpec((BE, w), lambda i: (i, 0))


def _full(r, c):
    return pl.BlockSpec((r, c), lambda i: (0, 0))


def kernel(fmess, bgraph, Wi, bi, Wo, bo, Wf, bf, Wu, bu):
    Wx = jnp.concatenate([Wi[:, :IN], Wo[:, :IN], Wu[:, :IN], Wf[:, :IN]], axis=0).T
    bcat = jnp.concatenate([bi, bo, bu, bf]).reshape(1, 4 * HID)
    Wh = jnp.concatenate([Wi[:, IN:], Wo[:, IN:], Wu[:, IN:]], axis=0).T
    WfhT = Wf[:, IN:].T
    idx = bgraph.reshape(-1)

    grid = (E // BE,)
    params = pltpu.CompilerParams(dimension_semantics=("parallel",))

    pio, pf, rec = pl.pallas_call(
        _phase_a_body,
        grid=grid,
        in_specs=[_rows(IN), _full(IN, 4 * HID), _full(1, 4 * HID), _full(HID, HID)],
        out_specs=[_rows(3 * HID), _rows(HID), _rows(REC)],
        out_shape=[jax.ShapeDtypeStruct((E, 3 * HID), F32),
                   jax.ShapeDtypeStruct((E, HID), F32),
                   jax.ShapeDtypeStruct((E, REC), F32)],
        compiler_params=params,
    )(fmess, Wx, bcat, WfhT)

    scg = _sc_gather()
    hs, ca = scg(rec, pf, idx)

    rec = pl.pallas_call(
        _phase_c_mid_body,
        grid=grid,
        in_specs=[_rows(3 * HID), _rows(HID), _rows(HID),
                  _full(HID, 3 * HID), _full(HID, HID)],
        out_specs=_rows(REC),
        out_shape=jax.ShapeDtypeStruct((E, REC), F32),
        compiler_params=params,
    )(pio, hs, ca, Wh, WfhT)

    hs, ca = scg(rec, pf, idx)

    h, c = pl.pallas_call(
        _phase_c_final_body,
        grid=grid,
        in_specs=[_rows(3 * HID), _rows(HID), _rows(HID), _full(HID, 3 * HID)],
        out_specs=[_rows(HID), _rows(HID)],
        out_shape=[jax.ShapeDtypeStruct((E, HID), F32),
                   jax.ShapeDtypeStruct((E, HID), F32)],
        compiler_params=params,
    )(pio, hs, ca, Wh)

    return (h, c)


# speedup vs baseline: 48.8532x; 2.2426x over previous
import functools

import jax
import jax.numpy as jnp
from jax import lax
from jax.experimental import pallas as pl
from jax.experimental.pallas import tpu as pltpu
from jax.experimental.pallas import tpu_sc as plsc

E = 160000
NNEI = 8
IN = 128
HID = 64
F32 = jnp.float32

BE = 2000
NC = 2
NS = 16
NW = NC * NS
PER_W = E // NW
CH = 8
NCHUNK = PER_W // CH
ROWS = CH * NNEI
LANES = 16
NV = HID // LANES
REC = 4 * HID


def _sig(x):
    return 1.0 / (1.0 + jnp.exp(-x))


def _keep_mask(nrows):
    pid = pl.program_id(0)
    rowids = lax.broadcasted_iota(jnp.int32, (nrows, 1), 0) + pid * nrows
    return (rowids != 0).astype(F32)


def _phase_a_body(f_ref, wx_ref, b_ref, wfh_ref, pio_ref, pf_ref, rec_ref):
    pre = jnp.dot(f_ref[...], wx_ref[...], preferred_element_type=F32) + b_ref[...]
    pio_ref[...] = pre[:, :3 * HID]
    pf_ref[...] = pre[:, 3 * HID:]
    i0 = _sig(pre[:, :HID])
    o0 = _sig(pre[:, HID:2 * HID])
    u0 = jnp.tanh(pre[:, 2 * HID:3 * HID])
    keep = _keep_mask(BE)
    c1 = i0 * u0 * keep
    h1 = o0 * jnp.tanh(c1) * keep
    g1 = jnp.dot(h1, wfh_ref[...], preferred_element_type=F32)
    rec_ref[...] = jnp.concatenate([h1, g1, c1, jnp.zeros((BE, HID), F32)], axis=1)


def _phase_c_mid_body(pio_ref, hs_ref, ca_ref, wh_ref, wfh_ref, rec_ref):
    t = pio_ref[...] + jnp.dot(hs_ref[...], wh_ref[...], preferred_element_type=F32)
    i = _sig(t[:, :HID])
    o = _sig(t[:, HID:2 * HID])
    u = jnp.tanh(t[:, 2 * HID:])
    keep = _keep_mask(BE)
    c = (i * u + ca_ref[...]) * keep
    h = o * jnp.tanh(c) * keep
    g = jnp.dot(h, wfh_ref[...], preferred_element_type=F32)
    rec_ref[...] = jnp.concatenate([h, g, c, jnp.zeros((BE, HID), F32)], axis=1)


def _phase_c_final_body(pio_ref, hs_ref, ca_ref, wh_ref, h_ref, c_ref):
    t = pio_ref[...] + jnp.dot(hs_ref[...], wh_ref[...], preferred_element_type=F32)
    i = _sig(t[:, :HID])
    o = _sig(t[:, HID:2 * HID])
    u = jnp.tanh(t[:, 2 * HID:])
    keep = _keep_mask(BE)
    c = (i * u + ca_ref[...]) * keep
    h = o * jnp.tanh(c) * keep
    h_ref[...] = h
    c_ref[...] = c


def _sc_body(rec_hbm, pref_hbm, idx_hbm, hs_hbm, ca_hbm,
             idx_all, rows0, rows1, pref0, pref1,
             hs0, hs1, ca0, ca1,
             gsem0, gsem1, psem0, psem1, ssem0, ssem1):
    wid = lax.axis_index("s") * NC + lax.axis_index("c")
    base = wid * PER_W
    rows = (rows0, rows1)
    pref = (pref0, pref1)
    hsb = (hs0, hs1)
    cab = (ca0, ca1)
    gsem = (gsem0, gsem1)
    psem = (psem0, psem1)
    ssem = (ssem0, ssem1)

    def g_copy(i, b):
        return pltpu.make_async_copy(
            rec_hbm.at[idx_all.at[pl.ds(i * ROWS, ROWS)]], rows[b], gsem[b])

    def p_copy(i, b):
        return pltpu.make_async_copy(
            pref_hbm.at[pl.ds(base + i * CH, CH)], pref[b], psem[b])

    def s_copies(i, b):
        m0 = base + i * CH
        return (pltpu.make_async_copy(hsb[b], hs_hbm.at[pl.ds(m0, CH)], ssem[b]),
                pltpu.make_async_copy(cab[b], ca_hbm.at[pl.ds(m0, CH)], ssem[b]))

    def compute(b):
        def msg_body(m, c2):
            r0 = m * NNEI
            for v in range(NV):
                sl = pl.ds(v * LANES, LANES)
                slg = pl.ds(HID + v * LANES, LANES)
                slc = pl.ds(2 * HID + v * LANES, LANES)
                p = pref[b][m, sl]
                hs_acc = rows[b][r0, sl]
                ca_acc = rows[b][r0, slc] / (1.0 + jnp.exp(-(p + rows[b][r0, slg])))
                for k in range(1, NNEI):
                    hs_acc = hs_acc + rows[b][r0 + k, sl]
                    ca_acc = ca_acc + rows[b][r0 + k, slc] / (
                        1.0 + jnp.exp(-(p + rows[b][r0 + k, slg])))
                hsb[b][m, sl] = hs_acc
                cab[b][m, sl] = ca_acc
            return c2

        lax.fori_loop(0, CH, msg_body, 0)

    pltpu.sync_copy(idx_hbm.at[pl.ds(base * NNEI, PER_W * NNEI)], idx_all)
    for b in (0, 1):
        g_copy(b, b).start()
        p_copy(b, b).start()

    def pair_body(pp, carry):
        for b in (0, 1):
            i = pp * 2 + b
            g_copy(i, b).wait()
            p_copy(i, b).wait()

            @pl.when(pp > 0)
            def _wait_store():
                sh, sc = s_copies(i, b)
                sh.wait()
                sc.wait()

            compute(b)
            sh, sc = s_copies(i, b)
            sh.start()
            sc.start()

            @pl.when(i + 2 < NCHUNK)
            def _prefetch():
                g_copy(i + 2, b).start()
                p_copy(i + 2, b).start()
        return carry

    lax.fori_loop(0, NCHUNK // 2, pair_body, 0)
    last = NCHUNK - 1
    g_copy(last, 0).wait()
    p_copy(last, 0).wait()
    sh, sc = s_copies(last, 0)
    sh.wait()
    sc.wait()
    compute(0)
    sh, sc = s_copies(last, 0)
    sh.start()
    sc.start()
    for b in (0, 1):
        sh, sc = s_copies(last, b)
        sh.wait()
        sc.wait()


@functools.cache
def _sc_gather():
    mesh = plsc.VectorSubcoreMesh(core_axis_name="c", subcore_axis_name="s")
    return pl.kernel(
        _sc_body,
        mesh=mesh,
        out_type=(jax.ShapeDtypeStruct((E, HID), F32),
                  jax.ShapeDtypeStruct((E, HID), F32)),
        scratch_types=[
            pltpu.VMEM((PER_W * NNEI,), jnp.int32),
            pltpu.VMEM((ROWS, REC), F32),
            pltpu.VMEM((ROWS, REC), F32),
            pltpu.VMEM((CH, HID), F32),
            pltpu.VMEM((CH, HID), F32),
            pltpu.VMEM((CH, HID), F32),
            pltpu.VMEM((CH, HID), F32),
            pltpu.VMEM((CH, HID), F32),
            pltpu.VMEM((CH, HID), F32),
            pltpu.SemaphoreType.DMA,
            pltpu.SemaphoreType.DMA,
            pltpu.SemaphoreType.DMA,
            pltpu.SemaphoreType.DMA,
            pltpu.SemaphoreType.DMA,
            pltpu.SemaphoreType.DMA,
        ],
    )


def _rows(w):
    return pl.BlockSpec((BE, w), lambda i: (i, 0))


def _full(r, c):
    return pl.BlockSpec((r, c), lambda i: (0, 0))


def kernel(fmess, bgraph, Wi, bi, Wo, bo, Wf, bf, Wu, bu):
    Wx = jnp.concatenate([Wi[:, :IN], Wo[:, :IN], Wu[:, :IN], Wf[:, :IN]], axis=0).T
    bcat = jnp.concatenate([bi, bo, bu, bf]).reshape(1, 4 * HID)
    Wh = jnp.concatenate([Wi[:, IN:], Wo[:, IN:], Wu[:, IN:]], axis=0).T
    WfhT = Wf[:, IN:].T
    idx = bgraph.reshape(-1)

    grid = (E // BE,)
    params = pltpu.CompilerParams(dimension_semantics=("parallel",))

    pio, pf, rec = pl.pallas_call(
        _phase_a_body,
        grid=grid,
        in_specs=[_rows(IN), _full(IN, 4 * HID), _full(1, 4 * HID), _full(HID, HID)],
        out_specs=[_rows(3 * HID), _rows(HID), _rows(REC)],
        out_shape=[jax.ShapeDtypeStruct((E, 3 * HID), F32),
                   jax.ShapeDtypeStruct((E, HID), F32),
                   jax.ShapeDtypeStruct((E, REC), F32)],
        compiler_params=params,
    )(fmess, Wx, bcat, WfhT)

    scg = _sc_gather()
    hs, ca = scg(rec, pf, idx)

    rec = pl.pallas_call(
        _phase_c_mid_body,
        grid=grid,
        in_specs=[_rows(3 * HID), _rows(HID), _rows(HID),
                  _full(HID, 3 * HID), _full(HID, HID)],
        out_specs=_rows(REC),
        out_shape=jax.ShapeDtypeStruct((E, REC), F32),
        compiler_params=params,
    )(pio, hs, ca, Wh, WfhT)

    hs, ca = scg(rec, pf, idx)

    h, c = pl.pallas_call(
        _phase_c_final_body,
        grid=grid,
        in_specs=[_rows(3 * HID), _rows(HID), _rows(HID), _full(HID, 3 * HID)],
        out_specs=[_rows(HID), _rows(HID)],
        out_shape=[jax.ShapeDtypeStruct((E, HID), F32),
                   jax.ShapeDtypeStruct((E, HID), F32)],
        compiler_params=params,
    )(pio, hs, ca, Wh)

    return (h, c)
